# baseline (speedup 1.0000x reference)
import functools

import jax
import jax.numpy as jnp
from jax import lax
from jax.experimental import pallas as pl
from jax.experimental.pallas import tpu as pltpu
from jax.experimental.pallas import tpu_sc as plsc

B, L, D = 4, 4096, 1024

_info = plsc.get_sparse_core_info()
NC, NS, LANES = _info.num_cores, _info.num_subcores, _info.num_lanes
NW = NC * NS
L_PER_W = L // NW
CHUNK = 8
N_STEPS = L_PER_W // CHUNK
N_GROUPS = CHUNK * D // LANES
GROUPS_PER_ROW = D // LANES
NBUF = 3
N_MAIN = (N_STEPS - 1) // NBUF

_mesh = plsc.VectorSubcoreMesh(core_axis_name="c", subcore_axis_name="s")

_scratch = (
    [pltpu.VMEM((CHUNK, D), jnp.float32) for _ in range(NBUF * B)]
    + [pltpu.VMEM((CHUNK, D), jnp.float32) for _ in range(NBUF)]
    + [pltpu.SemaphoreType.DMA for _ in range(3 * NBUF)]
)


@functools.partial(
    pl.kernel,
    mesh=_mesh,
    out_type=jax.ShapeDtypeStruct((B, L, D), jnp.float32),
    scratch_types=_scratch,
    compiler_params=pltpu.CompilerParams(use_tc_tiling_on_sc=True),
)
def _pos_emb_add(x_hbm, emb_hbm, out_hbm, *scratch):
    xv = [scratch[s * B:(s + 1) * B] for s in range(NBUF)]
    ev = scratch[NBUF * B:NBUF * B + NBUF]
    sems = scratch[NBUF * B + NBUF:]
    sem_xin = sems[0:NBUF]
    sem_ein = sems[NBUF:2 * NBUF]
    sem_xout = sems[2 * NBUF:3 * NBUF]

    wid = lax.axis_index("s") * NC + lax.axis_index("c")
    l_base = wid * L_PER_W

    def in_descs(step, slot):
        l0 = pl.multiple_of(l_base + step * CHUNK, CHUNK)
        descs = [pltpu.make_async_copy(
            emb_hbm.at[pl.ds(l0, CHUNK), :], ev[slot], sem_ein[slot])]
        for b in range(B):
            descs.append(pltpu.make_async_copy(
                x_hbm.at[b, pl.ds(l0, CHUNK), :], xv[slot][b],
                sem_xin[slot]))
        return descs

    def out_descs(step, slot):
        l0 = pl.multiple_of(l_base + step * CHUNK, CHUNK)
        return [pltpu.make_async_copy(
            xv[slot][b], out_hbm.at[b, pl.ds(l0, CHUNK), :], sem_xout[slot])
            for b in range(B)]

    def start(descs):
        for d in descs:
            d.start()

    def wait(descs):
        for d in descs:
            d.wait()

    def compute(slot):
        e_ref = ev[slot]
        x_refs = xv[slot]

        @plsc.parallel_loop(0, N_GROUPS, unroll=4)
        def _add(i):
            r = i // GROUPS_PER_ROW
            sl = pl.ds((i % GROUPS_PER_ROW) * LANES, LANES)
            e = e_ref[r, sl]
            for b in range(B):
                plsc.addupdate(x_refs[b].at[r, sl], e)

    for s in range(NBUF - 1):
        start(in_descs(s, s))

    for s in range(NBUF - 1):
        wait(in_descs(s, s))
        compute(s)
        start(out_descs(s, s))
        if s - 1 >= 0:
            wait(out_descs(s - 1, s - 1))
        start(in_descs(s + NBUF - 1, (s + NBUF - 1) % NBUF))

    HEAD = NBUF - 1
    N_MAIN = (N_STEPS - HEAD - (NBUF - 1)) // NBUF

    def main_body(g, carry):
        for j in range(NBUF):
            s = HEAD + g * NBUF + j
            slot = (HEAD + j) % NBUF
            wait(in_descs(s, slot))
            compute(slot)
            start(out_descs(s, slot))
            wait(out_descs(s - 1, (slot + NBUF - 1) % NBUF))
            start(in_descs(s + NBUF - 1, (slot + NBUF - 1) % NBUF))
        return carry

    lax.fori_loop(0, N_MAIN, main_body, 0)

    first_tail = HEAD + N_MAIN * NBUF
    for s in range(first_tail, N_STEPS):
        wait(in_descs(s, s % NBUF))
        compute(s % NBUF)
        start(out_descs(s, s % NBUF))
    for t in range(first_tail - 1, N_STEPS):
        wait(out_descs(t, t % NBUF))


def kernel(x, emb_table):
    return _pos_emb_add(x, emb_table)


# speedup vs baseline: 1.0052x; 1.0052x over previous
import functools

import jax
import jax.numpy as jnp
from jax import lax
from jax.experimental import pallas as pl
from jax.experimental.pallas import tpu as pltpu
from jax.experimental.pallas import tpu_sc as plsc

B, L, D = 4, 4096, 1024

_info = plsc.get_sparse_core_info()
NC, NS, LANES = _info.num_cores, _info.num_subcores, _info.num_lanes
NW = NC * NS
L_PER_W = L // NW
CHUNK = 8
N_STEPS = L_PER_W // CHUNK
N_GROUPS = CHUNK * D // LANES
GROUPS_PER_ROW = D // LANES
NBUF = 3
N_MAIN = (N_STEPS - 1) // NBUF

_mesh = plsc.VectorSubcoreMesh(core_axis_name="c", subcore_axis_name="s")

_scratch = (
    [pltpu.VMEM((CHUNK, D), jnp.float32) for _ in range(NBUF * B)]
    + [pltpu.VMEM((CHUNK, D), jnp.float32) for _ in range(NBUF)]
    + [pltpu.SemaphoreType.DMA for _ in range(3 * NBUF)]
)


@functools.partial(
    pl.kernel,
    mesh=_mesh,
    out_type=jax.ShapeDtypeStruct((B, L, D), jnp.float32),
    scratch_types=_scratch,
    compiler_params=pltpu.CompilerParams(use_tc_tiling_on_sc=True),
)
def _pos_emb_add(x_hbm, emb_hbm, out_hbm, *scratch):
    xv = [scratch[s * B:(s + 1) * B] for s in range(NBUF)]
    ev = scratch[NBUF * B:NBUF * B + NBUF]
    sems = scratch[NBUF * B + NBUF:]
    sem_xin = sems[0:NBUF]
    sem_ein = sems[NBUF:2 * NBUF]
    sem_xout = sems[2 * NBUF:3 * NBUF]

    wid = lax.axis_index("s") * NC + lax.axis_index("c")
    l_base = wid * L_PER_W

    def in_descs(step, slot):
        l0 = pl.multiple_of(l_base + step * CHUNK, CHUNK)
        descs = [pltpu.make_async_copy(
            emb_hbm.at[pl.ds(l0, CHUNK), :], ev[slot], sem_ein[slot])]
        for b in range(B):
            descs.append(pltpu.make_async_copy(
                x_hbm.at[b, pl.ds(l0, CHUNK), :], xv[slot][b],
                sem_xin[slot]))
        return descs

    def out_descs(step, slot):
        l0 = pl.multiple_of(l_base + step * CHUNK, CHUNK)
        return [pltpu.make_async_copy(
            xv[slot][b], out_hbm.at[b, pl.ds(l0, CHUNK), :], sem_xout[slot])
            for b in range(B)]

    def start(descs):
        for d in descs:
            d.start()

    def wait(descs):
        for d in descs:
            d.wait()

    def compute(slot):
        e_ref = ev[slot]
        x_refs = xv[slot]

        @plsc.parallel_loop(0, N_GROUPS, unroll=2)
        def _add(i):
            r = i // GROUPS_PER_ROW
            sl = pl.ds((i % GROUPS_PER_ROW) * LANES, LANES)
            e = e_ref[r, sl]
            for b in range(B):
                plsc.addupdate(x_refs[b].at[r, sl], e)

    for s in range(NBUF - 1):
        start(in_descs(s, s))

    for s in range(NBUF - 1):
        wait(in_descs(s, s))
        compute(s)
        start(out_descs(s, s))
        if s - 1 >= 0:
            wait(out_descs(s - 1, s - 1))
        start(in_descs(s + NBUF - 1, (s + NBUF - 1) % NBUF))

    HEAD = NBUF - 1
    N_MAIN = (N_STEPS - HEAD - (NBUF - 1)) // NBUF

    def main_body(g, carry):
        for j in range(NBUF):
            s = HEAD + g * NBUF + j
            slot = (HEAD + j) % NBUF
            wait(in_descs(s, slot))
            compute(slot)
            start(out_descs(s, slot))
            wait(out_descs(s - 1, (slot + NBUF - 1) % NBUF))
            start(in_descs(s + NBUF - 1, (slot + NBUF - 1) % NBUF))
        return carry

    lax.fori_loop(0, N_MAIN, main_body, 0)

    first_tail = HEAD + N_MAIN * NBUF
    for s in range(first_tail, N_STEPS):
        wait(in_descs(s, s % NBUF))
        compute(s % NBUF)
        start(out_descs(s, s % NBUF))
    for t in range(first_tail - 1, N_STEPS):
        wait(out_descs(t, t % NBUF))


def kernel(x, emb_table):
    return _pos_emb_add(x, emb_table)
